# Initial kernel scaffold; baseline (speedup 1.0000x reference)
import jax
import jax.numpy as jnp
from jax.experimental import pallas as pl


def kernel(drug_adj, circ_adj, x_drug, x_cir, params, drug_edge_index, circ_edge_index):
    raise NotImplementedError("write your pallas kernel here")


# speedup vs baseline: 1.1553x; 1.1553x over previous
import functools

import jax
import jax.numpy as jnp
from jax.experimental import pallas as pl
from jax.experimental.pallas import tpu as pltpu

N_NODE = 4096
F = 128
BLK = 512


def _fea_body(x1_ref, x2_ref, c1_ref, c2_ref, b_ref, o_ref):
    acc = jnp.dot(x1_ref[...], c1_ref[...], preferred_element_type=jnp.float32)
    acc += jnp.dot(x2_ref[...], c2_ref[...], preferred_element_type=jnp.float32)
    o_ref[...] = acc + b_ref[...]


def _fused_fea(x1, x2, cnn_W, cnn_b):
    c1 = cnn_W[:, 0, :, 0].T
    c2 = cnn_W[:, 1, :, 0].T
    b = cnn_b[None, :]
    n = x1.shape[0]
    return pl.pallas_call(
        _fea_body,
        grid=(n // BLK,),
        in_specs=[
            pl.BlockSpec((BLK, F), lambda i: (i, 0)),
            pl.BlockSpec((BLK, F), lambda i: (i, 0)),
            pl.BlockSpec((F, F), lambda i: (0, 0)),
            pl.BlockSpec((F, F), lambda i: (0, 0)),
            pl.BlockSpec((1, F), lambda i: (0, 0)),
        ],
        out_specs=pl.BlockSpec((BLK, F), lambda i: (i, 0)),
        out_shape=jax.ShapeDtypeStruct((n, F), jnp.float32),
    )(x1, x2, c1, c2, b)


def _mm_body(a_ref, b_ref, o_ref):
    o_ref[...] = jnp.dot(a_ref[...], b_ref[...].T,
                         preferred_element_type=jnp.float32)


def _final_mm(a, b):
    n = a.shape[0]
    return pl.pallas_call(
        _mm_body,
        grid=(n // BLK, n // BLK),
        in_specs=[
            pl.BlockSpec((BLK, F), lambda i, j: (i, 0)),
            pl.BlockSpec((BLK, F), lambda i, j: (j, 0)),
        ],
        out_specs=pl.BlockSpec((BLK, BLK), lambda i, j: (i, j)),
        out_shape=jax.ShapeDtypeStruct((n, n), jnp.float32),
    )(a, b)


def _layer(x, s, d, ew, deg_is, p, pre, n, heads):
    e_tot = s.shape[0]
    norm = deg_is[s] * ew * deg_is[d]
    agg_gcn = jax.ops.segment_sum(x[s] * norm[:, None], d, num_segments=n)
    gcn_out = agg_gcn @ pre['gcn_W'].T + pre['gcn_b']

    a_src = x @ pre['v_src']
    a_dst = x @ pre['v_dst']
    alpha = a_src[s] + a_dst[d]
    if pre['edge_c'] is not None:
        ea = jnp.concatenate(
            [ew[: e_tot - n], jnp.full((n,), pre['ea_fill'], ew.dtype)])
        alpha = alpha + ea[:, None] * pre['edge_c'][None, :]
    alpha = jax.nn.leaky_relu(alpha, negative_slope=0.2)
    ex = jnp.exp(alpha)
    denom = jax.ops.segment_sum(ex, d, num_segments=n)
    coef = ex / (denom[d] + 1e-16)
    agg = jax.ops.segment_sum(x[s][:, None, :] * coef[:, :, None], d,
                              num_segments=n)
    Wn = pre['gat_W'].reshape(heads, F, F)
    gat_out = jnp.mean(jnp.einsum('nhf,hgf->nhg', agg, Wn), axis=1) + pre['gat_b']
    return jax.nn.relu((gcn_out + gat_out) / 2.0)


def _graph_side(adj, x, ei, p, prefix, heads, n):
    s0, d0 = ei[0], ei[1]
    loop = jnp.arange(n, dtype=s0.dtype)
    s = jnp.concatenate([s0, loop])
    d = jnp.concatenate([d0, loop])
    ew_e = adj[s0, d0]
    ew = jnp.concatenate([ew_e, jnp.ones((n,), adj.dtype)])
    deg = jax.ops.segment_sum(ew, d, num_segments=n)
    deg_is = jnp.where(deg > 0, jax.lax.rsqrt(deg), 0.0)

    def mk_pre(gcn_W, gcn_b):
        gat_W = p[f'gat_{prefix}_W']
        att_src = p[f'gat_{prefix}_att_src'][0]
        att_dst = p[f'gat_{prefix}_att_dst'][0]
        Wn = gat_W.reshape(heads, F, F)
        v_src = jnp.einsum('hgf,hg->fh', Wn, att_src)
        v_dst = jnp.einsum('hgf,hg->fh', Wn, att_dst)
        pre = dict(gcn_W=gcn_W, gcn_b=gcn_b, gat_W=gat_W,
                   gat_b=p[f'gat_{prefix}_b'], v_src=v_src, v_dst=v_dst,
                   edge_c=None, ea_fill=None)
        if f'gat_{prefix}_lin_edge' in p:
            lin_e = p[f'gat_{prefix}_lin_edge'][:, 0].reshape(heads, F)
            att_e = p[f'gat_{prefix}_att_edge'][0]
            pre['edge_c'] = jnp.sum(lin_e * att_e, axis=1)
            pre['ea_fill'] = jnp.mean(ew_e)
        return pre

    pre1 = mk_pre(p[f'gcn_{prefix}1_W'], p[f'gcn_{prefix}1_b'])
    pre2 = mk_pre(p[f'gcn_{prefix}2_W'], p[f'gcn_{prefix}2_b'])
    x1 = _layer(x, s, d, ew, deg_is, p, pre1, n, heads)
    x2 = _layer(x1, s, d, ew, deg_is, p, pre2, n, heads)
    return x1, x2


def kernel(drug_adj, circ_adj, x_drug, x_cir, params, drug_edge_index,
           circ_edge_index):
    p = params
    x_d1, x_d2 = _graph_side(drug_adj, x_drug, drug_edge_index, p, 'd', 4,
                             N_NODE)
    x_c1, x_c2 = _graph_side(circ_adj, x_cir, circ_edge_index, p, 'c', 1,
                             N_NODE)
    drug_fea = _fused_fea(x_d1, x_d2, p['cnn_d_W'], p['cnn_d_b'])
    cir_fea = _fused_fea(x_c1, x_c2, p['cnn_c_W'], p['cnn_c_b'])
    return (_final_mm(cir_fea, drug_fea), drug_fea)


# speedup vs baseline: 2.6972x; 2.3346x over previous
import functools

import jax
import jax.numpy as jnp
from jax import lax
from jax.experimental import pallas as pl
from jax.experimental.pallas import tpu as pltpu
from jax.experimental.pallas import tpu_sc as plsc

N_NODE = 4096
F = 128
BLK = 512
NC, NS, L = 2, 16, 16
NW = NC * NS
TILE = 128


def _segsum_body(NT, x_hbm, s_hbm, d_hbm, w_hbm, out_hbm,
                 idx_s, idx_d, w_v, rows, scaled, acc, sem):
    cid = lax.axis_index("c")
    sid = lax.axis_index("s")
    wid = cid * NS + sid
    npp = N_NODE // NS

    pltpu.sync_copy(s_hbm.at[wid], idx_s)
    pltpu.sync_copy(d_hbm.at[wid], idx_d)
    pltpu.sync_copy(w_hbm.at[wid], w_v)

    def _zrow(e, _):
        for fb in range(F // L):
            scaled[e, pl.ds(fb * L, L)] = jnp.zeros((L,), jnp.float32)
        return 0
    lax.fori_loop(0, TILE, _zrow, 0)
    for z in range(npp // TILE):
        pltpu.sync_copy(scaled, acc.at[pl.ds(sid * npp + z * TILE, TILE)])
    plsc.subcore_barrier()

    def _tile(t, _):
        pltpu.async_copy(x_hbm.at[idx_s.at[t]], rows, sem).wait()

        def _grp(i, _):
            wvec = w_v[t, pl.ds(i * L, L)]
            for lane in range(L):
                wspl = wvec.at[jnp.full((L,), lane, jnp.int32)].get(
                    mode='promise_in_bounds')
                e = i * L + lane
                for fb in range(F // L):
                    sl = pl.ds(fb * L, L)
                    scaled[e, sl] = rows[e, sl] * wspl
            return 0
        lax.fori_loop(0, TILE // L, _grp, 0)
        pltpu.sync_copy(scaled, acc.at[idx_d.at[t]], add=True)
        return 0
    lax.fori_loop(0, NT, _tile, 0)

    plsc.subcore_barrier()

    row0 = cid * N_NODE + sid * npp
    pltpu.sync_copy(acc.at[pl.ds(sid * npp, npp)],
                    out_hbm.at[pl.ds(pl.multiple_of(row0, npp), npp)])


def make_segsum(e_pad):
    n_tiles = e_pad // TILE
    assert n_tiles % NW == 0
    NT = n_tiles // NW
    mesh = plsc.VectorSubcoreMesh(core_axis_name="c", subcore_axis_name="s",
                                  num_cores=NC, num_subcores=NS)
    scratch = [
        pltpu.VMEM((NT, TILE), jnp.int32),
        pltpu.VMEM((NT, TILE), jnp.int32),
        pltpu.VMEM((NT, TILE), jnp.float32),
        pltpu.VMEM((TILE, F), jnp.float32),
        pltpu.VMEM((TILE, F), jnp.float32),
        pltpu.VMEM_SHARED((N_NODE, F), jnp.float32),
        pltpu.SemaphoreType.DMA,
    ]
    return pl.kernel(
        functools.partial(_segsum_body, NT),
        out_type=jax.ShapeDtypeStruct((NC * N_NODE, F), jnp.float32),
        mesh=mesh,
        scratch_types=scratch,
    )


def _fea_body(x1_ref, x2_ref, c1_ref, c2_ref, b_ref, o_ref):
    acc = jnp.dot(x1_ref[...], c1_ref[...],
                  preferred_element_type=jnp.float32)
    acc += jnp.dot(x2_ref[...], c2_ref[...],
                   preferred_element_type=jnp.float32)
    o_ref[...] = acc + b_ref[...]


def _fused_fea(x1, x2, cnn_W, cnn_b):
    c1 = cnn_W[:, 0, :, 0].T
    c2 = cnn_W[:, 1, :, 0].T
    b = cnn_b[None, :]
    n = x1.shape[0]
    return pl.pallas_call(
        _fea_body,
        grid=(n // BLK,),
        in_specs=[
            pl.BlockSpec((BLK, F), lambda i: (i, 0)),
            pl.BlockSpec((BLK, F), lambda i: (i, 0)),
            pl.BlockSpec((F, F), lambda i: (0, 0)),
            pl.BlockSpec((F, F), lambda i: (0, 0)),
            pl.BlockSpec((1, F), lambda i: (0, 0)),
        ],
        out_specs=pl.BlockSpec((BLK, F), lambda i: (i, 0)),
        out_shape=jax.ShapeDtypeStruct((n, F), jnp.float32),
    )(x1, x2, c1, c2, b)


def _mm_body(a_ref, b_ref, o_ref):
    o_ref[...] = jnp.dot(a_ref[...], b_ref[...].T,
                         preferred_element_type=jnp.float32)


def _final_mm(a, b):
    n = a.shape[0]
    return pl.pallas_call(
        _mm_body,
        grid=(n // BLK, n // BLK),
        in_specs=[
            pl.BlockSpec((BLK, F), lambda i, j: (i, 0)),
            pl.BlockSpec((BLK, F), lambda i, j: (j, 0)),
        ],
        out_specs=pl.BlockSpec((BLK, BLK), lambda i, j: (i, j)),
        out_shape=jax.ShapeDtypeStruct((n, n), jnp.float32),
    )(a, b)


def _graph_side(adj, x, ei, p, prefix, heads, n):
    s0, d0 = ei[0], ei[1]
    loop = jnp.arange(n, dtype=s0.dtype)
    s = jnp.concatenate([s0, loop])
    d = jnp.concatenate([d0, loop])
    ew_e = adj[s0, d0]
    ew = jnp.concatenate([ew_e, jnp.ones((n,), adj.dtype)])
    deg = jax.ops.segment_sum(ew, d, num_segments=n)
    dis = jnp.where(deg > 0, 1.0 / jnp.sqrt(deg), 0.0)
    norm = dis[s] * ew * dis[d]
    e_tot = s.shape[0]
    n_tiles = e_tot // TILE
    NT = n_tiles // NW
    s3 = s.reshape(NW, NT, TILE)
    d3 = d.reshape(NW, NT, TILE)
    seg = make_segsum(e_tot)

    def agg(X, w):
        halves = seg(X, s3, d3, w.reshape(NW, NT, TILE)).reshape(NC, n, F)
        return halves[0] + halves[1]

    gat_W = p[f'gat_{prefix}_W']
    att_src = p[f'gat_{prefix}_att_src']
    att_dst = p[f'gat_{prefix}_att_dst']

    def layer(x, gcn_W, gcn_b):
        xw_g = x @ gcn_W.T
        xw = (x @ gat_W.T).reshape(n, heads, F)
        a_src = jnp.sum(xw * att_src, axis=-1)
        a_dst = jnp.sum(xw * att_dst, axis=-1)
        alpha = a_src[s] + a_dst[d]
        if f'gat_{prefix}_lin_edge' in p:
            ea = ew_e[:, None]
            fill = jnp.mean(ea, axis=0, keepdims=True)
            ea_full = jnp.concatenate([ea, jnp.tile(fill, (n, 1))], axis=0)
            e = (ea_full @ p[f'gat_{prefix}_lin_edge'].T).reshape(
                -1, heads, F)
            alpha = alpha + jnp.sum(e * p[f'gat_{prefix}_att_edge'], axis=-1)
        alpha = jax.nn.leaky_relu(alpha, negative_slope=0.2)
        amax = jax.ops.segment_max(alpha, d, num_segments=n)
        amax = jnp.where(jnp.isfinite(amax), amax, 0.0)
        ex = jnp.exp(alpha - amax[d])
        denom = jax.ops.segment_sum(ex, d, num_segments=n)
        coef = ex / (denom[d] + 1e-16)

        gcn_out = agg(xw_g, norm) + gcn_b
        gat_sum = agg(xw[:, 0, :], coef[:, 0])
        for h in range(1, heads):
            gat_sum = gat_sum + agg(xw[:, h, :], coef[:, h])
        gat_out = gat_sum / heads + p[f'gat_{prefix}_b']
        return jax.nn.relu((gcn_out + gat_out) / 2.0)

    x1 = layer(x, p[f'gcn_{prefix}1_W'], p[f'gcn_{prefix}1_b'])
    x2 = layer(x1, p[f'gcn_{prefix}2_W'], p[f'gcn_{prefix}2_b'])
    return x1, x2


def kernel(drug_adj, circ_adj, x_drug, x_cir, params, drug_edge_index,
           circ_edge_index):
    p = params
    x_d1, x_d2 = _graph_side(drug_adj, x_drug, drug_edge_index, p, 'd', 4,
                             N_NODE)
    x_c1, x_c2 = _graph_side(circ_adj, x_cir, circ_edge_index, p, 'c', 1,
                             N_NODE)
    drug_fea = _fused_fea(x_d1, x_d2, p['cnn_d_W'], p['cnn_d_b'])
    cir_fea = _fused_fea(x_c1, x_c2, p['cnn_c_W'], p['cnn_c_b'])
    return (_final_mm(cir_fea, drug_fea), drug_fea)
